# Initial kernel scaffold; baseline (speedup 1.0000x reference)
import jax
import jax.numpy as jnp
from jax.experimental import pallas as pl


def kernel(edge_index, rel_type, norm, W0, W1, W2):
    raise NotImplementedError("write your pallas kernel here")


# speedup vs baseline: 5.0490x; 5.0490x over previous
import functools

import jax
import jax.numpy as jnp
from jax import lax
from jax.experimental import pallas as pl
from jax.experimental.pallas import tpu as pltpu
from jax.experimental.pallas import tpu_sc as plsc

_N = 10000
_H = 128
_OUT = 16
_R = 16
_E = 320000

_NC = 2
_NS = 16
_NW = _NC * _NS
_EP = 10240
_E_PAD = _NW * _EP
_IVL = 128
_NV = _EP // _IVL
_CHUNK_IV = 2
_C = _CHUNK_IV * _IVL
_NCHUNK = _NV // _CHUNK_IV
_NPAD = 10240
_RPT = _NPAD // _NS
_ZR = 128


def _sc_pass_body(mode, d, table, srcm, relm, dstm, normm, out,
                  idx_v, rel_v, dst_v, norm_v, rows_v, acc_ref, sem):
    cid = lax.axis_index("c")
    sid = lax.axis_index("s")
    wid = sid * _NC + cid
    ng = d // 16

    pltpu.sync_copy(normm.at[wid], norm_v)

    def _zero_row(r, _):
        for g in range(ng):
            rows_v[r, pl.ds(g * 16, 16)] = jnp.zeros((16,), jnp.float32)
        return 0
    lax.fori_loop(0, _ZR, _zero_row, 0)

    for j in range(_RPT // _ZR):
        pltpu.sync_copy(rows_v.at[pl.ds(0, _ZR)],
                        acc_ref.at[pl.ds(sid * _RPT + j * _ZR, _ZR)])
    plsc.subcore_barrier()

    def _chunk(i, _):
        pltpu.sync_copy(srcm.at[wid, pl.ds(_CHUNK_IV * i, _CHUNK_IV)], idx_v)
        pltpu.sync_copy(relm.at[wid, pl.ds(_CHUNK_IV * i, _CHUNK_IV)], rel_v)
        pltpu.sync_copy(dstm.at[wid, pl.ds(_CHUNK_IV * i, _CHUNK_IV)], dst_v)

        for k in range(_CHUNK_IV):
            for g in range(8):
                sl = pl.ds(g * 16, 16)
                s = idx_v[k, sl]
                rl = rel_v[k, sl]
                if mode == 0:
                    idx_v[k, sl] = rl * _N + s
                else:
                    idx_v[k, sl] = s * _R + rl

        descs = [
            pltpu.async_copy(table.at[idx_v.at[k]],
                             rows_v.at[pl.ds(k * _IVL, _IVL)], sem)
            for k in range(_CHUNK_IV)
        ]
        for dsc in descs:
            dsc.wait()

        def _scale(q, _):
            nv16 = norm_v[pl.ds(i * _C + q * 16, 16)]
            for l in range(16):
                nv = jnp.full((16,), nv16[l], jnp.float32)
                for g in range(ng):
                    sl = pl.ds(g * 16, 16)
                    rows_v[q * 16 + l, sl] = rows_v[q * 16 + l, sl] * nv
            return 0
        lax.fori_loop(0, _C // 16, _scale, 0)

        for k in range(_CHUNK_IV):
            pltpu.sync_copy(rows_v.at[pl.ds(k * _IVL, _IVL)],
                            acc_ref.at[dst_v.at[k]],
                            add=True)
        return 0
    lax.fori_loop(0, _NCHUNK, _chunk, 0)

    plsc.subcore_barrier()
    pltpu.sync_copy(acc_ref.at[pl.ds(sid * _RPT, _RPT)],
                    out.at[cid, pl.ds(sid * _RPT, _RPT)])


def _sc_pass(table, srcm, relm, dstm, normm, mode):
    d = table.shape[1]
    body = functools.partial(_sc_pass_body, mode, d)
    return pl.kernel(
        body,
        out_type=jax.ShapeDtypeStruct((_NC, _NPAD, d), jnp.float32),
        mesh=plsc.VectorSubcoreMesh(core_axis_name="c", subcore_axis_name="s"),
        scratch_types=[
            pltpu.VMEM((_CHUNK_IV, _IVL), jnp.int32),
            pltpu.VMEM((_CHUNK_IV, _IVL), jnp.int32),
            pltpu.VMEM((_CHUNK_IV, _IVL), jnp.int32),
            pltpu.VMEM((_EP,), jnp.float32),
            pltpu.VMEM((_C, d), jnp.float32),
            pltpu.VMEM_SHARED((_NPAD, d), jnp.float32),
            pltpu.SemaphoreType.DMA,
        ],
        compiler_params=pltpu.CompilerParams(use_tc_tiling_on_sc=False),
    )(table, srcm, relm, dstm, normm)


def _mm_body(p_ref, w_ref, o_ref):
    h = jnp.maximum(p_ref[0] + p_ref[1], 0.0)
    o_ref[...] = jnp.dot(h, w_ref[...], preferred_element_type=jnp.float32)


def _tc_relu_matmul(p, w):
    n = p.shape[1]
    f = w.shape[1]
    rb = 1000
    fb = min(f, 256)
    grid = (n // rb, f // fb)
    return pl.pallas_call(
        _mm_body,
        grid=grid,
        in_specs=[
            pl.BlockSpec((2, rb, _H), lambda i, j: (0, i, 0)),
            pl.BlockSpec((_H, fb), lambda i, j: (0, j)),
        ],
        out_specs=pl.BlockSpec((rb, fb), lambda i, j: (i, j)),
        out_shape=jax.ShapeDtypeStruct((n, f), jnp.float32),
    )(p, w)


def _softmax_body(p_ref, o_ref):
    x = p_ref[0] + p_ref[1]
    m = jnp.max(x, axis=1, keepdims=True)
    e = jnp.exp(x - m)
    o_ref[...] = e / jnp.sum(e, axis=1, keepdims=True)


def _tc_softmax(p):
    n = p.shape[1]
    d = p.shape[2]
    rb = 1000
    return pl.pallas_call(
        _softmax_body,
        grid=(n // rb,),
        in_specs=[pl.BlockSpec((2, rb, d), lambda i: (0, i, 0))],
        out_specs=pl.BlockSpec((rb, d), lambda i: (i, 0)),
        out_shape=jax.ShapeDtypeStruct((n, d), jnp.float32),
    )(p)


def kernel(edge_index, rel_type, norm, W0, W1, W2):
    src = edge_index[0]
    dst = edge_index[1]
    nrm = norm[:, 0]

    pad = _E_PAD - _E
    srcm = jnp.pad(src, (0, pad)).reshape(_NW, _NV, _IVL)
    relm = jnp.pad(rel_type, (0, pad)).reshape(_NW, _NV, _IVL)
    dstm = jnp.pad(dst, (0, pad)).reshape(_NW, _NV, _IVL)
    normm = jnp.pad(nrm, (0, pad)).reshape(_NW, _EP)

    p0 = _sc_pass(W0.reshape(_R * _N, _H), srcm, relm, dstm, normm,
                  mode=0)[:, :_N]

    xw1 = _tc_relu_matmul(p0, W1.transpose(1, 0, 2).reshape(_H, _R * _H))
    p1 = _sc_pass(xw1.reshape(_N * _R, _H), srcm, relm, dstm, normm,
                  mode=1)[:, :_N]

    xw2 = _tc_relu_matmul(p1, W2.transpose(1, 0, 2).reshape(_H, _R * _OUT))
    p2 = _sc_pass(xw2.reshape(_N * _R, _OUT), srcm, relm, dstm, normm,
                  mode=1)[:, :_N]

    return _tc_softmax(p2)
